# baseline (speedup 1.0000x reference)
import functools

import jax
import jax.numpy as jnp
from jax import lax
from jax.experimental import pallas as pl
from jax.experimental.pallas import tpu as pltpu
from jax.experimental.pallas import tpu_sc as plsc

NC = 2
NS = 16
L = 16
NW = NC * NS
IDX_CHUNK = 128


@functools.cache
def _build(B, D, V, A):
    assert B % (8 * NW) == 0
    bpw = B // NW
    assert bpw % IDX_CHUNK == 0
    nch = bpw // IDX_CHUNK
    ngr = bpw // L

    mesh = plsc.VectorSubcoreMesh(core_axis_name="c", subcore_axis_name="s")

    @functools.partial(
        pl.kernel,
        out_type=jax.ShapeDtypeStruct((B, D), jnp.float32),
        mesh=mesh,
        scratch_types=[
            pltpu.VMEM((bpw,), jnp.int32),
            pltpu.VMEM((bpw,), jnp.int32),
            pltpu.VMEM((bpw,), jnp.int32),
            pltpu.VMEM((bpw,), jnp.int32),
            pltpu.VMEM((bpw,), jnp.int32),
            pltpu.VMEM((bpw, D), jnp.float32),
            pltpu.VMEM((bpw, D), jnp.float32),
            pltpu.SemaphoreType.DMA,
            pltpu.SemaphoreType.DMA,
            pltpu.SemaphoreType.DMA,
        ],
        compiler_params=pltpu.CompilerParams(
            needs_layout_passes=False, use_tc_tiling_on_sc=False),
    )
    def bank(utab, atab, ids_hbm, aids_hbm, mwords_hbm, out_hbm,
             ids_v, aids_v, cids_v, widx_v, words_v, arows_v, urows_v,
             sem_a, sem_m, sem_u):
        wid = lax.axis_index("s") * NC + lax.axis_index("c")
        base = wid * bpw

        pltpu.sync_copy(ids_hbm.at[pl.ds(base, bpw)], ids_v)
        pltpu.sync_copy(aids_hbm.at[pl.ds(base, bpw)], aids_v)

        for g in range(ngr):
            sl = pl.ds(g * L, L)
            v = jnp.minimum(jnp.maximum(ids_v[sl], 0), V - 1)
            cids_v[sl] = v
            widx_v[sl] = v >> 2

        copies = []
        for j in range(nch):
            sl = pl.ds(j * IDX_CHUNK, IDX_CHUNK)
            copies.append(
                pltpu.async_copy(atab.at[aids_v.at[sl]], arows_v.at[sl], sem_a))
            copies.append(
                pltpu.async_copy(mwords_hbm.at[widx_v.at[sl]], words_v.at[sl],
                                 sem_m))
        for c in copies:
            c.wait()

        acc = jnp.zeros((L,), jnp.int32)
        for g in range(ngr):
            sl = pl.ds(g * L, L)
            acc = acc | ((words_v[sl] >> ((cids_v[sl] & 3) * 8)) & 0xFF)
        any_known = acc[0]
        for i in range(1, L):
            any_known = any_known | acc[i]

        @pl.when(any_known != 0)
        def _fallback():
            ucopies = [
                pltpu.async_copy(
                    utab.at[cids_v.at[pl.ds(j * IDX_CHUNK, IDX_CHUNK)]],
                    urows_v.at[pl.ds(j * IDX_CHUNK, IDX_CHUNK)], sem_u)
                for j in range(nch)
            ]
            for c in ucopies:
                c.wait()
            lanes = lax.iota(jnp.int32, L)

            def merge_group(g, _):
                sl = pl.ds(g * L, L)
                cid = cids_v[sl]
                known = ((words_v[sl] >> ((cid & 3) * 8)) & 0xFF) != 0
                rows = g * L + lanes

                def merge_col(col, _):
                    cvec = jnp.full((L,), col, jnp.int32)
                    u = plsc.load_gather(urows_v, [rows, cvec])
                    plsc.store_scatter(arows_v, [rows, cvec], u, mask=known)
                    return 0

                lax.fori_loop(0, D, merge_col, 0)
                return 0

            lax.fori_loop(0, ngr, merge_group, 0)

        pltpu.sync_copy(arows_v, out_hbm.at[pl.ds(base, bpw)])

    return bank


def kernel(user_table, archetype_table, user_ids, archetype_ids,
           known_user_mask, batch_size):
    V, D = user_table.shape
    A = archetype_table.shape[0]
    B = user_ids.shape[0]
    assert V % 4 == 0

    ids = user_ids.astype(jnp.int32)
    aids = archetype_ids.astype(jnp.int32)
    m8 = known_user_mask.reshape(-1, 4).astype(jnp.int32)
    mwords = (m8[:, 0] | (m8[:, 1] << 8) | (m8[:, 2] << 16) | (m8[:, 3] << 24))

    return _build(B, D, V, A)(user_table, archetype_table, ids, aids, mwords)


# speedup vs baseline: 12.0025x; 12.0025x over previous
import functools

import jax
import jax.numpy as jnp
from jax import lax
from jax.experimental import pallas as pl
from jax.experimental.pallas import tpu as pltpu
from jax.experimental.pallas import tpu_sc as plsc

NC = 2
NS = 16
L = 16
NW = NC * NS
IDX_CHUNK = 128

_SC_PARAMS = pltpu.CompilerParams(
    needs_layout_passes=False, use_tc_tiling_on_sc=False)


@functools.cache
def _build_fast(B, D, V, A):
    assert B % (8 * NW) == 0
    bpw = B // NW
    ngr = bpw // L
    mch = -(-V // NW)
    mch += (-mch) % 64
    nmg = mch // 64

    mesh = plsc.VectorSubcoreMesh(core_axis_name="c", subcore_axis_name="s")

    @functools.partial(
        pl.kernel,
        out_type=(jax.ShapeDtypeStruct((B, D), jnp.float32),
                  jax.ShapeDtypeStruct((NW * L,), jnp.int32)),
        mesh=mesh,
        scratch_types=[
            pltpu.VMEM((bpw,), jnp.int32),
            pltpu.VMEM((A, D), jnp.float32),
            pltpu.VMEM((mch,), jnp.uint8),
            pltpu.VMEM((L,), jnp.int32),
            pltpu.VMEM((bpw, D), jnp.float32),
        ],
        compiler_params=_SC_PARAMS,
    )
    def fast(m8_hbm, atab, aids_hbm, out_hbm, part_hbm,
             aids_v, atab_v, m_v, part_v, rows_v):
        wid = lax.axis_index("s") * NC + lax.axis_index("c")
        base = wid * bpw

        pltpu.sync_copy(aids_hbm.at[pl.ds(base, bpw)], aids_v)
        pltpu.sync_copy(atab, atab_v)
        mstart = jnp.minimum(wid * mch, V - mch)
        pltpu.sync_copy(m8_hbm.at[pl.ds(mstart, mch)], m_v)

        acc8 = jnp.zeros((4 * L,), jnp.uint8)
        for i in range(nmg):
            acc8 = acc8 | m_v[pl.ds(i * 64, 64)]
        part_v[...] = plsc.bitcast(acc8, jnp.int32)
        pltpu.sync_copy(part_v, part_hbm.at[pl.ds(wid * L, L)])

        lanes = lax.iota(jnp.int32, L)

        def build_group(g, _):
            aid = aids_v[pl.ds(g * L, L)]
            rows = g * L + lanes
            for c in range(D):
                cvec = jnp.full((L,), c, jnp.int32)
                v = plsc.load_gather(atab_v, [aid, cvec])
                plsc.store_scatter(rows_v, [rows, cvec], v)
            return 0

        lax.fori_loop(0, ngr, build_group, 0)

        pltpu.sync_copy(rows_v, out_hbm.at[pl.ds(base, bpw)])

    return fast


@functools.cache
def _build_general(B, D, V, A):
    bpw = B // NW
    nch = bpw // IDX_CHUNK
    ngr = bpw // L

    mesh = plsc.VectorSubcoreMesh(core_axis_name="c", subcore_axis_name="s")

    @functools.partial(
        pl.kernel,
        out_type=jax.ShapeDtypeStruct((B, D), jnp.float32),
        mesh=mesh,
        scratch_types=[
            pltpu.VMEM((bpw,), jnp.int32),
            pltpu.VMEM((bpw,), jnp.int32),
            pltpu.VMEM((bpw,), jnp.int32),
            pltpu.VMEM((bpw,), jnp.int32),
            pltpu.VMEM((bpw,), jnp.int32),
            pltpu.VMEM((bpw, D), jnp.float32),
            pltpu.VMEM((bpw, D), jnp.float32),
            pltpu.SemaphoreType.DMA,
            pltpu.SemaphoreType.DMA,
            pltpu.SemaphoreType.DMA,
        ],
        compiler_params=_SC_PARAMS,
    )
    def bank(utab, atab, ids_hbm, aids_hbm, mwords_hbm, out_hbm,
             ids_v, aids_v, cids_v, widx_v, words_v, arows_v, urows_v,
             sem_a, sem_m, sem_u):
        wid = lax.axis_index("s") * NC + lax.axis_index("c")
        base = wid * bpw

        pltpu.sync_copy(ids_hbm.at[pl.ds(base, bpw)], ids_v)
        pltpu.sync_copy(aids_hbm.at[pl.ds(base, bpw)], aids_v)

        for g in range(ngr):
            sl = pl.ds(g * L, L)
            v = jnp.minimum(jnp.maximum(ids_v[sl], 0), V - 1)
            cids_v[sl] = v
            widx_v[sl] = v >> 2

        copies = []
        for j in range(nch):
            sl = pl.ds(j * IDX_CHUNK, IDX_CHUNK)
            copies.append(
                pltpu.async_copy(atab.at[aids_v.at[sl]], arows_v.at[sl], sem_a))
            copies.append(
                pltpu.async_copy(mwords_hbm.at[widx_v.at[sl]], words_v.at[sl],
                                 sem_m))
            copies.append(
                pltpu.async_copy(utab.at[cids_v.at[sl]], urows_v.at[sl], sem_u))
        for c in copies:
            c.wait()

        lanes = lax.iota(jnp.int32, L)

        def merge_group(g, _):
            sl = pl.ds(g * L, L)
            cid = cids_v[sl]
            known = ((words_v[sl] >> ((cid & 3) * 8)) & 0xFF) != 0
            rows = g * L + lanes

            def merge_col(col, _):
                cvec = jnp.full((L,), col, jnp.int32)
                u = plsc.load_gather(urows_v, [rows, cvec])
                plsc.store_scatter(arows_v, [rows, cvec], u, mask=known)
                return 0

            lax.fori_loop(0, D, merge_col, 0)
            return 0

        lax.fori_loop(0, ngr, merge_group, 0)

        pltpu.sync_copy(arows_v, out_hbm.at[pl.ds(base, bpw)])

    return bank


def kernel(user_table, archetype_table, user_ids, archetype_ids,
           known_user_mask, batch_size):
    V, D = user_table.shape
    A = archetype_table.shape[0]
    B = user_ids.shape[0]
    assert V % 4 == 0

    aids = archetype_ids.astype(jnp.int32)
    m8 = known_user_mask.astype(jnp.uint8)

    out_fast, partials = _build_fast(B, D, V, A)(
        m8, archetype_table, aids)
    any_known = jnp.any(partials != 0)

    def cold(_):
        ids = user_ids.astype(jnp.int32)
        mw8 = m8.reshape(-1, 4).astype(jnp.int32)
        mwords = (mw8[:, 0] | (mw8[:, 1] << 8) | (mw8[:, 2] << 16)
                  | (mw8[:, 3] << 24))
        return _build_general(B, D, V, A)(
            user_table, archetype_table, ids, aids, mwords)

    return lax.cond(any_known, cold, lambda x: x, out_fast)


# speedup vs baseline: 21.1555x; 1.7626x over previous
import functools

import jax
import jax.numpy as jnp
from jax import lax
from jax.experimental import pallas as pl
from jax.experimental.pallas import tpu as pltpu
from jax.experimental.pallas import tpu_sc as plsc

NC = 2
NS = 16
L = 16
NW = NC * NS
IDX_CHUNK = 128

_SC_PARAMS = pltpu.CompilerParams(
    needs_layout_passes=False, use_tc_tiling_on_sc=False)


@functools.cache
def _build_fast(B, D, V, A):
    assert B % (8 * NW) == 0
    bpw = B // NW
    ngr = bpw // L
    mch = -(-V // NW)
    mch += (-mch) % 64
    nmg = mch // 64

    mesh = plsc.VectorSubcoreMesh(core_axis_name="c", subcore_axis_name="s")

    @functools.partial(
        pl.kernel,
        out_type=(jax.ShapeDtypeStruct((B, D), jnp.float32),
                  jax.ShapeDtypeStruct((NW * L,), jnp.int32)),
        mesh=mesh,
        scratch_types=[
            pltpu.VMEM((bpw,), jnp.int32),
            pltpu.VMEM_SHARED((A, D), jnp.float32),
            pltpu.VMEM((mch,), jnp.uint8),
            pltpu.VMEM((L,), jnp.int32),
            pltpu.VMEM((bpw, D), jnp.float32),
            pltpu.SemaphoreType.DMA,
        ],
        compiler_params=_SC_PARAMS,
    )
    def fast(m8_hbm, atab, aids_hbm, out_hbm, part_hbm,
             aids_v, atab_sh, m_v, part_v, rows_v, sem):
        sid = lax.axis_index("s")
        wid = sid * NC + lax.axis_index("c")
        base = wid * bpw

        @pl.when(sid == 0)
        def _stage():
            pltpu.sync_copy(atab, atab_sh)

        pltpu.sync_copy(aids_hbm.at[pl.ds(base, bpw)], aids_v)
        mstart = jnp.minimum(wid * mch, V - mch)
        pltpu.sync_copy(m8_hbm.at[pl.ds(mstart, mch)], m_v)

        acc8 = jnp.zeros((4 * L,), jnp.uint8)
        for i in range(nmg):
            acc8 = acc8 | m_v[pl.ds(i * 64, 64)]
        part_v[...] = plsc.bitcast(acc8, jnp.int32)
        pltpu.sync_copy(part_v, part_hbm.at[pl.ds(wid * L, L)])

        plsc.subcore_barrier()
        copies = []
        for j in range(bpw // IDX_CHUNK):
            sl = pl.ds(j * IDX_CHUNK, IDX_CHUNK)
            copies.append(
                pltpu.async_copy(atab_sh.at[aids_v.at[sl]], rows_v.at[sl],
                                 sem))
        for c in copies:
            c.wait()

        pltpu.sync_copy(rows_v, out_hbm.at[pl.ds(base, bpw)])

    return fast


@functools.cache
def _build_general(B, D, V, A):
    bpw = B // NW
    nch = bpw // IDX_CHUNK
    ngr = bpw // L

    mesh = plsc.VectorSubcoreMesh(core_axis_name="c", subcore_axis_name="s")

    @functools.partial(
        pl.kernel,
        out_type=jax.ShapeDtypeStruct((B, D), jnp.float32),
        mesh=mesh,
        scratch_types=[
            pltpu.VMEM((bpw,), jnp.int32),
            pltpu.VMEM((bpw,), jnp.int32),
            pltpu.VMEM((bpw,), jnp.int32),
            pltpu.VMEM((bpw,), jnp.int32),
            pltpu.VMEM((bpw,), jnp.int32),
            pltpu.VMEM((bpw, D), jnp.float32),
            pltpu.VMEM((bpw, D), jnp.float32),
            pltpu.SemaphoreType.DMA,
            pltpu.SemaphoreType.DMA,
            pltpu.SemaphoreType.DMA,
        ],
        compiler_params=_SC_PARAMS,
    )
    def bank(utab, atab, ids_hbm, aids_hbm, mwords_hbm, out_hbm,
             ids_v, aids_v, cids_v, widx_v, words_v, arows_v, urows_v,
             sem_a, sem_m, sem_u):
        wid = lax.axis_index("s") * NC + lax.axis_index("c")
        base = wid * bpw

        pltpu.sync_copy(ids_hbm.at[pl.ds(base, bpw)], ids_v)
        pltpu.sync_copy(aids_hbm.at[pl.ds(base, bpw)], aids_v)

        for g in range(ngr):
            sl = pl.ds(g * L, L)
            v = jnp.minimum(jnp.maximum(ids_v[sl], 0), V - 1)
            cids_v[sl] = v
            widx_v[sl] = v >> 2

        copies = []
        for j in range(nch):
            sl = pl.ds(j * IDX_CHUNK, IDX_CHUNK)
            copies.append(
                pltpu.async_copy(atab.at[aids_v.at[sl]], arows_v.at[sl], sem_a))
            copies.append(
                pltpu.async_copy(mwords_hbm.at[widx_v.at[sl]], words_v.at[sl],
                                 sem_m))
            copies.append(
                pltpu.async_copy(utab.at[cids_v.at[sl]], urows_v.at[sl], sem_u))
        for c in copies:
            c.wait()

        lanes = lax.iota(jnp.int32, L)

        def merge_group(g, _):
            sl = pl.ds(g * L, L)
            cid = cids_v[sl]
            known = ((words_v[sl] >> ((cid & 3) * 8)) & 0xFF) != 0
            rows = g * L + lanes

            def merge_col(col, _):
                cvec = jnp.full((L,), col, jnp.int32)
                u = plsc.load_gather(urows_v, [rows, cvec])
                plsc.store_scatter(arows_v, [rows, cvec], u, mask=known)
                return 0

            lax.fori_loop(0, D, merge_col, 0)
            return 0

        lax.fori_loop(0, ngr, merge_group, 0)

        pltpu.sync_copy(arows_v, out_hbm.at[pl.ds(base, bpw)])

    return bank


def kernel(user_table, archetype_table, user_ids, archetype_ids,
           known_user_mask, batch_size):
    V, D = user_table.shape
    A = archetype_table.shape[0]
    B = user_ids.shape[0]
    assert V % 4 == 0

    aids = archetype_ids.astype(jnp.int32)
    m8 = known_user_mask.astype(jnp.uint8)

    out_fast, partials = _build_fast(B, D, V, A)(
        m8, archetype_table, aids)
    any_known = jnp.any(partials != 0)

    def cold(_):
        ids = user_ids.astype(jnp.int32)
        mw8 = m8.reshape(-1, 4).astype(jnp.int32)
        mwords = (mw8[:, 0] | (mw8[:, 1] << 8) | (mw8[:, 2] << 16)
                  | (mw8[:, 3] << 24))
        return _build_general(B, D, V, A)(
            user_table, archetype_table, ids, aids, mwords)

    return lax.cond(any_known, cold, lambda x: x, out_fast)
